# baseline (speedup 1.0000x reference)
import functools

import jax
import jax.numpy as jnp
from jax import lax
from jax.experimental import pallas as pl
from jax.experimental.pallas import tpu as pltpu
from jax.experimental.pallas import tpu_sc as plsc

PRE_SEQ_LEN = 128
ROW_DIM = 98304
BATCH = 4
B_FLAT = 512

NUM_CORES = 2
NUM_SUBCORES = 16
ROWS_PER_TEC = B_FLAT // NUM_SUBCORES
A_ROWS = 8
B_ROWS = ROWS_PER_TEC - A_ROWS
HALF_COLS = ROW_DIM // NUM_CORES
CHUNK = 1536
NUM_CHUNKS = HALF_COLS // CHUNK
STAGE_ROWS = PRE_SEQ_LEN // NUM_SUBCORES
UNROLL = 4


def make_sc_gather():
    mesh = plsc.VectorSubcoreMesh(core_axis_name="c", subcore_axis_name="s")

    @functools.partial(
        pl.kernel,
        out_type=jax.ShapeDtypeStruct((B_FLAT, ROW_DIM), jnp.float32),
        mesh=mesh,
        scratch_types=[
            pltpu.VMEM((ROWS_PER_TEC,), jnp.int32),
            pltpu.VMEM((2, B_ROWS, CHUNK), jnp.float32),
            pltpu.VMEM_SHARED((4, PRE_SEQ_LEN, CHUNK), jnp.float32),
            pltpu.SemaphoreType.DMA,
            pltpu.SemaphoreType.DMA,
            pltpu.SemaphoreType.DMA,
            pltpu.SemaphoreType.DMA,
            pltpu.SemaphoreType.DMA,
            pltpu.SemaphoreType.DMA,
            pltpu.SemaphoreType.DMA,
            pltpu.SemaphoreType.DMA,
        ],
    )
    def sc_gather(
        idx_hbm, table_hbm, out_hbm,
        idx_v, tb, sp,
        st_sem, xb_sem, aw0, aw1, aw2, aw3, bw0, bw1,
    ):
        sc = lax.axis_index("c")
        tec = lax.axis_index("s")
        row0 = tec * ROWS_PER_TEC
        col0 = sc * HALF_COLS
        srow = tec * STAGE_ROWS
        pltpu.sync_copy(idx_hbm.at[pl.ds(row0, ROWS_PER_TEC)], idx_v)
        idx_lo = idx_v[pl.ds(0, 16)]
        idx_hi = idx_v[pl.ds(16, 16)]
        idx_s = [idx_lo[i] for i in range(16)] + [idx_hi[i] for i in range(16)]

        aws = (aw0, aw1, aw2, aw3)
        bws = (bw0, bw1)

        def stage(c, k4):
            return pltpu.make_async_copy(
                table_hbm.at[pl.ds(srow, STAGE_ROWS), pl.ds(col0 + c * CHUNK, CHUNK)],
                sp.at[k4, pl.ds(srow, STAGE_ROWS)],
                st_sem,
            )

        def a_write(c, k4, i):
            return pltpu.make_async_copy(
                sp.at[k4, pl.ds(idx_s[i], 1)],
                out_hbm.at[pl.ds(row0 + i, 1), pl.ds(col0 + c * CHUNK, CHUNK)],
                aws[k4],
            )

        def b_fill(k4, q2, i):
            return pltpu.make_async_copy(
                sp.at[k4, pl.ds(idx_s[A_ROWS + i], 1)],
                tb.at[q2, pl.ds(i, 1)],
                xb_sem,
            )

        def b_write(c, q2):
            return pltpu.make_async_copy(
                tb.at[q2],
                out_hbm.at[
                    pl.ds(row0 + A_ROWS, B_ROWS),
                    pl.ds(col0 + c * CHUNK, CHUNK),
                ],
                bws[q2],
            )

        def step(c, u):
            k4 = u % 4
            q2 = u % 2
            stage(c, k4).wait()

            @pl.when(c >= 2)
            def _drain_a():
                for i in range(A_ROWS):
                    a_write(c - 2, (u - 2) % 4, i).wait()

            plsc.subcore_barrier()

            @pl.when(c + 1 < NUM_CHUNKS)
            def _next_stage():
                stage(c + 1, (u + 1) % 4).start()

            for i in range(A_ROWS):
                a_write(c, k4, i).start()

            @pl.when(c >= 2)
            def _drain_b():
                b_write(c - 2, q2).wait()

            for i in range(B_ROWS):
                b_fill(k4, q2, i).start()
            for i in range(B_ROWS):
                b_fill(k4, q2, i).wait()
            b_write(c, q2).start()

        stage(0, 0).start()

        def body(j, _):
            for u in range(UNROLL):
                step(UNROLL * j + u, u)
            return _

        lax.fori_loop(0, NUM_CHUNKS // UNROLL, body, 0, unroll=False)
        for c in (NUM_CHUNKS - 2, NUM_CHUNKS - 1):
            for i in range(A_ROWS):
                a_write(c, c % 4, i).wait()
            b_write(c, c % 2).wait()

    return sc_gather


def kernel(prefix, embedding):
    idx = prefix.reshape(B_FLAT)
    out = make_sc_gather()(idx, embedding)
    return out.reshape(BATCH, PRE_SEQ_LEN, ROW_DIM)


# speedup vs baseline: 1.0231x; 1.0231x over previous
import functools

import jax
import jax.numpy as jnp
from jax import lax
from jax.experimental import pallas as pl
from jax.experimental.pallas import tpu as pltpu
from jax.experimental.pallas import tpu_sc as plsc

PRE_SEQ_LEN = 128
ROW_DIM = 98304
BATCH = 4
B_FLAT = 512

NUM_CORES = 2
NUM_SUBCORES = 16
ROWS_PER_TEC = B_FLAT // NUM_SUBCORES
A_ROWS = 16
B_ROWS = ROWS_PER_TEC - A_ROWS
HALF_COLS = ROW_DIM // NUM_CORES
CHUNK = 2048
NUM_CHUNKS = HALF_COLS // CHUNK
SUB = 2048
NUM_SUBS = CHUNK // SUB
STAGE_ROWS = PRE_SEQ_LEN // NUM_SUBCORES
UNROLL = 6


def make_sc_gather():
    mesh = plsc.VectorSubcoreMesh(core_axis_name="c", subcore_axis_name="s")

    @functools.partial(
        pl.kernel,
        out_type=jax.ShapeDtypeStruct((B_FLAT, ROW_DIM), jnp.float32),
        mesh=mesh,
        scratch_types=[
            pltpu.VMEM((ROWS_PER_TEC,), jnp.int32),
            pltpu.VMEM((B_ROWS, SUB), jnp.float32),
            pltpu.VMEM((B_ROWS, SUB), jnp.float32),
            pltpu.VMEM_SHARED((PRE_SEQ_LEN, CHUNK), jnp.float32),
            pltpu.VMEM_SHARED((PRE_SEQ_LEN, CHUNK), jnp.float32),
            pltpu.VMEM_SHARED((PRE_SEQ_LEN, CHUNK), jnp.float32),
            pltpu.SemaphoreType.DMA,
            pltpu.SemaphoreType.DMA,
            pltpu.SemaphoreType.DMA,
            pltpu.SemaphoreType.DMA,
            pltpu.SemaphoreType.DMA,
            pltpu.SemaphoreType.DMA,
            pltpu.SemaphoreType.DMA,
        ],
    )
    def sc_gather(
        idx_hbm, table_hbm, out_hbm,
        idx_v, tb0, tb1, sp0, sp1, sp2,
        st_sem, xb_sem, aw0, aw1, aw2, bw0, bw1,
    ):
        sc = lax.axis_index("c")
        tec = lax.axis_index("s")
        row0 = tec * ROWS_PER_TEC
        col0 = sc * HALF_COLS
        srow = tec * STAGE_ROWS
        pltpu.sync_copy(idx_hbm.at[pl.ds(row0, ROWS_PER_TEC)], idx_v)
        idx_lo = idx_v[pl.ds(0, 16)]
        idx_hi = idx_v[pl.ds(16, 16)]
        idx_s = [idx_lo[i] for i in range(16)] + [idx_hi[i] for i in range(16)]

        sps = (sp0, sp1, sp2)
        aws = (aw0, aw1, aw2)
        tbs = (tb0, tb1)
        bws = (bw0, bw1)

        def stage(c, k3):
            return pltpu.make_async_copy(
                table_hbm.at[pl.ds(srow, STAGE_ROWS), pl.ds(col0 + c * CHUNK, CHUNK)],
                sps[k3].at[pl.ds(srow, STAGE_ROWS)],
                st_sem,
            )

        def a_write(c, k3, i):
            return pltpu.make_async_copy(
                sps[k3].at[pl.ds(idx_s[i], 1)],
                out_hbm.at[pl.ds(row0 + i, 1), pl.ds(col0 + c * CHUNK, CHUNK)],
                aws[k3],
            )

        def b_fill(k3, s, q2, i):
            return pltpu.make_async_copy(
                sps[k3].at[pl.ds(idx_s[A_ROWS + i], 1), pl.ds(s * SUB, SUB)],
                tbs[q2].at[pl.ds(i, 1)],
                xb_sem,
            )

        def b_write(c, s, q2):
            return pltpu.make_async_copy(
                tbs[q2],
                out_hbm.at[
                    pl.ds(row0 + A_ROWS, B_ROWS),
                    pl.ds(col0 + c * CHUNK + s * SUB, SUB),
                ],
                bws[q2],
            )

        def step(c, u):
            k3 = u % 3
            stage(c, k3).wait()

            @pl.when(c >= 2)
            def _drain_a():
                for i in range(A_ROWS):
                    a_write(c - 2, (u - 2) % 3, i).wait()

            plsc.subcore_barrier()

            @pl.when(c + 1 < NUM_CHUNKS)
            def _next_stage():
                stage(c + 1, (u + 1) % 3).start()

            for i in range(A_ROWS):
                a_write(c, k3, i).start()

            for s in range(NUM_SUBS):
                t = u * NUM_SUBS + s
                q2 = t % 2

                ss = (s - 2) % NUM_SUBS
                borrow = 0 if s >= 2 else (2 - s + NUM_SUBS - 1) // NUM_SUBS

                @pl.when(c * NUM_SUBS + s >= 2)
                def _drain_b(c=c, ss=ss, borrow=borrow, q2=q2):
                    b_write(c - borrow, ss, q2).wait()

                for i in range(B_ROWS):
                    b_fill(k3, s, q2, i).start()
                for i in range(B_ROWS):
                    b_fill(k3, s, q2, i).wait()
                b_write(c, s, q2).start()

        stage(0, 0).start()

        def body(j, _):
            for u in range(UNROLL):
                step(UNROLL * j + u, u)
            return _

        lax.fori_loop(0, NUM_CHUNKS // UNROLL, body, 0, unroll=False)
        for i in range(A_ROWS):
            a_write(NUM_CHUNKS - 2, (NUM_CHUNKS - 2) % 3, i).wait()
        for i in range(A_ROWS):
            a_write(NUM_CHUNKS - 1, (NUM_CHUNKS - 1) % 3, i).wait()
        total_subs = NUM_CHUNKS * NUM_SUBS
        for g in (total_subs - 2, total_subs - 1):
            cg, sg = g // NUM_SUBS, g % NUM_SUBS
            qg = ((cg % UNROLL) * NUM_SUBS + sg) % 2
            b_write(cg, sg, qg).wait()

    return sc_gather


def kernel(prefix, embedding):
    idx = prefix.reshape(B_FLAT)
    out = make_sc_gather()(idx, embedding)
    return out.reshape(BATCH, PRE_SEQ_LEN, ROW_DIM)
